# baseline (device time: 25877 ns/iter reference)
import jax
import jax.numpy as jnp
from jax import lax
from jax.experimental import pallas as pl
from jax.experimental.pallas import tpu as pltpu

N_DEV = 16
R_HOPS = 4
L_HOPS = 3
H = 2


def _ring_pos(my):
    z = my // 4
    j = my % 4
    return jnp.where(
        j == 0, (N_DEV - z) % N_DEV,
        jnp.where(j == 1, 1 + z, jnp.where(j == 2, 8 - z, 9 + z)),
    )


def _ring_dev(q):
    q = q % N_DEV
    return jnp.where(
        (q >= 1) & (q <= 4), 4 * (q - 1) + 1,
        jnp.where(
            (q >= 5) & (q <= 8), 4 * (8 - q) + 2,
            jnp.where((q >= 9) & (q <= 12), 4 * (q - 9) + 3,
                      4 * ((N_DEV - q) % N_DEV)),
        ),
    )


def kernel(A, B):
    m_per, k = A.shape
    n = B.shape[1]
    mh = m_per // H

    def body(a_ref, b_ref, out_ref, cro, clo, cra, cla, stage,
             sa_s, sa_r, sro_s, sro_r, slo_s, slo_r,
             sra_s, sra_r, sla_s, sla_r, out_sems):
        my = lax.axis_index("i")
        p = _ring_pos(my)
        right = _ring_dev(p + 1)
        left = _ring_dev(p - 1)
        anti = _ring_dev(p + 8)

        b = (b_ref[:, :] * (4.0 / 127.0)).astype(jnp.bfloat16)
        a_q = jnp.clip(
            jnp.rint(a_ref[:, :] * (127.0 / 4.0)), -127.0, 127.0
        ).astype(jnp.int8)
        for h in range(H):
            cro[0, h, :, :] = a_q[h * mh:(h + 1) * mh, :]
            clo[0, h, :, :] = a_q[h * mh:(h + 1) * mh, :]

        def mk(src, dst, ssem, rsem, dev):
            return pltpu.make_async_remote_copy(
                src_ref=src, dst_ref=dst, send_sem=ssem, recv_sem=rsem,
                device_id=(dev,), device_id_type=pl.DeviceIdType.MESH,
            )

        def r_own(s, h):
            return mk(cro.at[s, h], cro.at[s + 1, h],
                      sro_s.at[s, h], sro_r.at[s, h], right)

        def l_own(s, h):
            return mk(clo.at[s, h], clo.at[s + 1, h],
                      slo_s.at[s, h], slo_r.at[s, h], left)

        def r_anti(s, h):
            return mk(cra.at[s, h], cra.at[s + 1, h],
                      sra_s.at[s, h], sra_r.at[s, h], right)

        def l_anti(s, h):
            src = cra.at[0, h] if s == 0 else cla.at[s, h]
            return mk(src, cla.at[s + 1, h],
                      sla_s.at[s, h], sla_r.at[s, h], left)

        def a_send(h):
            return mk(cro.at[0, h], cra.at[0, h], sa_s.at[h], sa_r.at[h],
                      anti)

        _slot_ctr = [0]
        _slot_writes = []

        def out_copy(slot, h, origin):
            return pltpu.make_async_copy(
                stage.at[slot, h],
                out_ref.at[pl.ds(origin * m_per + h * mh, mh), :],
                out_sems.at[slot, h],
            )

        def compute(ref, s, origin_pos):
            origin = _ring_dev(origin_pos)
            slot = _slot_ctr[0]
            _slot_ctr[0] += 1
            _slot_writes.append(origin_pos)
            for h in range(H):
                stage[slot, h, :, :] = jnp.dot(
                    ref[s, h, :, :].astype(jnp.bfloat16), b,
                    preferred_element_type=jnp.float32,
                ).astype(jnp.bfloat16)
                out_copy(slot, h, origin).start()

        barrier_sem = pltpu.get_barrier_semaphore()
        for nbr in [left, right, anti]:
            pl.semaphore_signal(
                barrier_sem, inc=1,
                device_id=(nbr,), device_id_type=pl.DeviceIdType.MESH,
            )
        pl.semaphore_wait(barrier_sem, 3)

        for h in range(H):
            a_send(h).start()
            r_own(0, h).start()
            l_own(0, h).start()
        compute(cro, 0, p)

        for h in range(H):
            a_send(h).wait_recv()
            r_anti(0, h).start()
            l_anti(0, h).start()
        compute(cra, 0, p + 8)

        for s in range(1, R_HOPS + 1):
            for h in range(H):
                r_own(s - 1, h).wait_recv()
                if s < R_HOPS:
                    r_own(s, h).start()
            if s <= L_HOPS:
                for h in range(H):
                    l_own(s - 1, h).wait_recv()
                    if s < L_HOPS:
                        l_own(s, h).start()
            compute(cro, s, p - s)
            if s <= L_HOPS:
                compute(clo, s, p + s)

            for h in range(H):
                r_anti(s - 1, h).wait_recv()
                if s < R_HOPS:
                    r_anti(s, h).start()
            if s <= L_HOPS:
                for h in range(H):
                    l_anti(s - 1, h).wait_recv()
                    if s < L_HOPS:
                        l_anti(s, h).start()
            compute(cra, s, p + 8 - s)
            if s <= L_HOPS:
                compute(cla, s, p + 8 + s)

        for slot, origin_pos in enumerate(_slot_writes):
            origin = _ring_dev(origin_pos)
            for h in range(H):
                out_copy(slot, h, origin).wait()
        for h in range(H):
            a_send(h).wait_send()
        for s in range(R_HOPS):
            for h in range(H):
                r_own(s, h).wait_send()
                r_anti(s, h).wait_send()
        for s in range(L_HOPS):
            for h in range(H):
                l_own(s, h).wait_send()
                l_anti(s, h).wait_send()

    return pl.pallas_call(
        body,
        out_shape=jax.ShapeDtypeStruct((N_DEV * m_per, n), jnp.bfloat16),
        in_specs=[
            pl.BlockSpec(memory_space=pltpu.VMEM),
            pl.BlockSpec(memory_space=pltpu.VMEM),
        ],
        out_specs=pl.BlockSpec(memory_space=pl.ANY),
        scratch_shapes=[
            pltpu.VMEM((R_HOPS + 1, H, mh, k), jnp.int8),
            pltpu.VMEM((L_HOPS + 1, H, mh, k), jnp.int8),
            pltpu.VMEM((R_HOPS + 1, H, mh, k), jnp.int8),
            pltpu.VMEM((L_HOPS + 1, H, mh, k), jnp.int8),
            pltpu.VMEM((N_DEV, H, mh, n), jnp.bfloat16),
            pltpu.SemaphoreType.DMA((H,)),
            pltpu.SemaphoreType.DMA((H,)),
            pltpu.SemaphoreType.DMA((R_HOPS, H)),
            pltpu.SemaphoreType.DMA((R_HOPS, H)),
            pltpu.SemaphoreType.DMA((L_HOPS, H)),
            pltpu.SemaphoreType.DMA((L_HOPS, H)),
            pltpu.SemaphoreType.DMA((R_HOPS, H)),
            pltpu.SemaphoreType.DMA((R_HOPS, H)),
            pltpu.SemaphoreType.DMA((L_HOPS, H)),
            pltpu.SemaphoreType.DMA((L_HOPS, H)),
            pltpu.SemaphoreType.DMA((N_DEV, H)),
        ],
        compiler_params=pltpu.CompilerParams(collective_id=0),
    )(A, B)


# device time: 23756 ns/iter; 1.0893x vs baseline; 1.0893x over previous
import jax
import jax.numpy as jnp
from jax import lax
from jax.experimental import pallas as pl
from jax.experimental.pallas import tpu as pltpu

N_DEV = 16
R_HOPS = 4
L_HOPS = 3
H = 2


def _ring_pos(my):
    z = my // 4
    j = my % 4
    return jnp.where(
        j == 0, (N_DEV - z) % N_DEV,
        jnp.where(j == 1, 1 + z, jnp.where(j == 2, 8 - z, 9 + z)),
    )


def _ring_dev(q):
    q = q % N_DEV
    return jnp.where(
        (q >= 1) & (q <= 4), 4 * (q - 1) + 1,
        jnp.where(
            (q >= 5) & (q <= 8), 4 * (8 - q) + 2,
            jnp.where((q >= 9) & (q <= 12), 4 * (q - 9) + 3,
                      4 * ((N_DEV - q) % N_DEV)),
        ),
    )


def kernel(A, B):
    m_per, k = A.shape
    n = B.shape[1]
    mh = m_per // H

    def body(a_ref, b_ref, out_ref, cro, clo, cra, cla,
             sa_s, sa_r, sro_s, sro_r, slo_s, slo_r,
             sra_s, sra_r, sla_s, sla_r):
        my = lax.axis_index("i")
        p = _ring_pos(my)
        right = _ring_dev(p + 1)
        left = _ring_dev(p - 1)
        anti = _ring_dev(p + 8)

        b = (b_ref[:, :] * (4.0 / 127.0)).astype(jnp.bfloat16)
        a_q = jnp.clip(
            jnp.rint(a_ref[:, :] * (127.0 / 4.0)), -127.0, 127.0
        ).astype(jnp.int8)
        for h in range(H):
            cro[0, h, :, :] = a_q[h * mh:(h + 1) * mh, :]
            clo[0, h, :, :] = a_q[h * mh:(h + 1) * mh, :]

        def mk(src, dst, ssem, rsem, dev):
            return pltpu.make_async_remote_copy(
                src_ref=src, dst_ref=dst, send_sem=ssem, recv_sem=rsem,
                device_id=(dev,), device_id_type=pl.DeviceIdType.MESH,
            )

        def r_own(s, h):
            return mk(cro.at[s, h], cro.at[s + 1, h],
                      sro_s.at[s, h], sro_r.at[s, h], right)

        def l_own(s, h):
            return mk(clo.at[s, h], clo.at[s + 1, h],
                      slo_s.at[s, h], slo_r.at[s, h], left)

        def r_anti(s, h):
            return mk(cra.at[s, h], cra.at[s + 1, h],
                      sra_s.at[s, h], sra_r.at[s, h], right)

        def l_anti(s, h):
            src = cra.at[0, h] if s == 0 else cla.at[s, h]
            return mk(src, cla.at[s + 1, h],
                      sla_s.at[s, h], sla_r.at[s, h], left)

        def a_send(h):
            return mk(cro.at[0, h], cra.at[0, h], sa_s.at[h], sa_r.at[h],
                      anti)

        def compute(ref, s, origin_pos):
            origin = _ring_dev(origin_pos)
            for h in range(H):
                out_ref[pl.ds(origin * m_per + h * mh, mh), :] = jnp.dot(
                    ref[s, h, :, :].astype(jnp.bfloat16), b,
                    preferred_element_type=jnp.float32,
                ).astype(jnp.bfloat16)

        barrier_sem = pltpu.get_barrier_semaphore()
        for nbr in [left, right, anti]:
            pl.semaphore_signal(
                barrier_sem, inc=1,
                device_id=(nbr,), device_id_type=pl.DeviceIdType.MESH,
            )
        pl.semaphore_wait(barrier_sem, 3)

        for h in range(H):
            a_send(h).start()
            r_own(0, h).start()
            l_own(0, h).start()
        compute(cro, 0, p)

        for h in range(H):
            a_send(h).wait_recv()
            r_anti(0, h).start()
            l_anti(0, h).start()
        compute(cra, 0, p + 8)

        for s in range(1, R_HOPS + 1):
            for h in range(H):
                r_own(s - 1, h).wait_recv()
                if s < R_HOPS:
                    r_own(s, h).start()
            if s <= L_HOPS:
                for h in range(H):
                    l_own(s - 1, h).wait_recv()
                    if s < L_HOPS:
                        l_own(s, h).start()
            compute(cro, s, p - s)
            if s <= L_HOPS:
                compute(clo, s, p + s)

            for h in range(H):
                r_anti(s - 1, h).wait_recv()
                if s < R_HOPS:
                    r_anti(s, h).start()
            if s <= L_HOPS:
                for h in range(H):
                    l_anti(s - 1, h).wait_recv()
                    if s < L_HOPS:
                        l_anti(s, h).start()
            compute(cra, s, p + 8 - s)
            if s <= L_HOPS:
                compute(cla, s, p + 8 + s)

        for h in range(H):
            a_send(h).wait_send()
        for s in range(R_HOPS):
            for h in range(H):
                r_own(s, h).wait_send()
                r_anti(s, h).wait_send()
        for s in range(L_HOPS):
            for h in range(H):
                l_own(s, h).wait_send()
                l_anti(s, h).wait_send()

    return pl.pallas_call(
        body,
        out_shape=jax.ShapeDtypeStruct((N_DEV * m_per, n), jnp.bfloat16),
        in_specs=[
            pl.BlockSpec(memory_space=pltpu.VMEM),
            pl.BlockSpec(memory_space=pltpu.VMEM),
        ],
        out_specs=pl.BlockSpec(memory_space=pltpu.VMEM),
        scratch_shapes=[
            pltpu.VMEM((R_HOPS + 1, H, mh, k), jnp.int8),
            pltpu.VMEM((L_HOPS + 1, H, mh, k), jnp.int8),
            pltpu.VMEM((R_HOPS + 1, H, mh, k), jnp.int8),
            pltpu.VMEM((L_HOPS + 1, H, mh, k), jnp.int8),
            pltpu.SemaphoreType.DMA((H,)),
            pltpu.SemaphoreType.DMA((H,)),
            pltpu.SemaphoreType.DMA((R_HOPS, H)),
            pltpu.SemaphoreType.DMA((R_HOPS, H)),
            pltpu.SemaphoreType.DMA((L_HOPS, H)),
            pltpu.SemaphoreType.DMA((L_HOPS, H)),
            pltpu.SemaphoreType.DMA((R_HOPS, H)),
            pltpu.SemaphoreType.DMA((R_HOPS, H)),
            pltpu.SemaphoreType.DMA((L_HOPS, H)),
            pltpu.SemaphoreType.DMA((L_HOPS, H)),
        ],
        compiler_params=pltpu.CompilerParams(collective_id=0),
    )(A, B)
